# baseline (speedup 1.0000x reference)
import functools

import jax
import jax.numpy as jnp
from jax import lax
from jax.experimental import pallas as pl
from jax.experimental.pallas import tpu as pltpu
from jax.experimental.pallas import tpu_sc as plsc

N = 10000
E = 320000
H = 128
BE = 2000

NC = 2
NS = 16
NW = NC * NS
PER_W = E // NW
K = 40

_sc_mesh = plsc.VectorSubcoreMesh(core_axis_name="c", subcore_axis_name="s")


def _make_geom_body(per_w, ch):
    def body(g_hbm, src_hbm, dst_hbm, d_hbm,
             srcv, dstv, gs0, gs1, gd0, gd1, dbuf,
             semS0, semS1, semD0, semD1, semW):
        gs = (gs0, gs1)
        gd = (gd0, gd1)
        semS = (semS0, semS1)
        semD = (semD0, semD1)
        c = lax.axis_index("c")
        s = lax.axis_index("s")
        wid = c * NS + s
        base = wid * per_w
        pltpu.sync_copy(src_hbm.at[pl.ds(base, per_w)], srcv)
        pltpu.sync_copy(dst_hbm.at[pl.ds(base, per_w)], dstv)

        def fire(k, b):
            o = k * K
            pltpu.async_copy(g_hbm.at[srcv.at[pl.ds(o, K)]], gs[b], semS[b])
            pltpu.async_copy(g_hbm.at[dstv.at[pl.ds(o, K)]], gd[b], semD[b])

        def process(j, b):
            pltpu.make_async_copy(
                g_hbm.at[srcv.at[pl.ds(0, K)]], gs[b], semS[b]).wait()
            pltpu.make_async_copy(
                g_hbm.at[dstv.at[pl.ds(0, K)]], gd[b], semD[b]).wait()

            @pl.when(j > 0)
            def _():
                pltpu.make_async_copy(
                    dbuf, d_hbm.at[pl.ds(base, K)], semW).wait()

            def row(r, _):
                sl = pl.ds(0, 16)
                dbuf[r, sl] = gs[b][r, sl] - gd[b][r, sl]
                return 0

            lax.fori_loop(0, K, row, 0)
            pltpu.async_copy(dbuf, d_hbm.at[pl.ds(base + j * K, K)], semW)

        fire(0, 0)

        def outer(g, _):
            for bb in (0, 1):
                j = 2 * g + bb

                @pl.when(j + 1 < ch)
                def _():
                    fire(j + 1, 1 - bb)

                @pl.when(j < ch)
                def _():
                    process(j, bb)
            return 0

        lax.fori_loop(0, (ch + 1) // 2, outer, 0)
        pltpu.make_async_copy(dbuf, d_hbm.at[pl.ds(base, K)], semW).wait()

    return body


def _sc_geom(g128, src, dst):
    f32 = jnp.float32
    ne = src.shape[0]
    per_w = ne // NW
    ch = per_w // K
    k = pl.kernel(
        _make_geom_body(per_w, ch),
        out_type=jax.ShapeDtypeStruct((ne, H), f32),
        mesh=_sc_mesh,
        scratch_types=[
            pltpu.VMEM((per_w,), jnp.int32),
            pltpu.VMEM((per_w,), jnp.int32),
            pltpu.VMEM((K, H), f32),
            pltpu.VMEM((K, H), f32),
            pltpu.VMEM((K, H), f32),
            pltpu.VMEM((K, H), f32),
            pltpu.VMEM((K, H), f32),
        ] + [pltpu.SemaphoreType.DMA] * 5,
    )
    return k(g128, src, dst)


def _make_gather_body(per_w, ch):
    def body(c_hbm, src_hbm, dst_hbm, pq_hbm, srcv, dstv,
             cs0, cs1, cd0, cd1, pqb, semS0, semS1, semD0, semD1, semW):
        cs = (cs0, cs1)
        cd = (cd0, cd1)
        semS = (semS0, semS1)
        semD = (semD0, semD1)
        c = lax.axis_index("c")
        s = lax.axis_index("s")
        wid = c * NS + s
        base = wid * per_w
        pltpu.sync_copy(src_hbm.at[pl.ds(base, per_w)], srcv)
        pltpu.sync_copy(dst_hbm.at[pl.ds(base, per_w)], dstv)

        def fire(k, b):
            o = k * K
            pltpu.async_copy(c_hbm.at[srcv.at[pl.ds(o, K)]], cs[b], semS[b])
            pltpu.async_copy(c_hbm.at[dstv.at[pl.ds(o, K)]], cd[b], semD[b])

        def process(j, b):
            pltpu.make_async_copy(
                c_hbm.at[srcv.at[pl.ds(0, K)]], cs[b], semS[b]).wait()
            pltpu.make_async_copy(
                c_hbm.at[dstv.at[pl.ds(0, K)]], cd[b], semD[b]).wait()

            @pl.when(j > 0)
            def _():
                pltpu.make_async_copy(
                    pqb, pq_hbm.at[pl.ds(base, K)], semW).wait()

            def row(r, _):
                for cc in range(H // 16):
                    sl = pl.ds(cc * 16, 16)
                    sl2 = pl.ds(H + cc * 16, 16)
                    pqb[r, sl] = cd[b][r, sl] + cs[b][r, sl2]
                    pqb[r, sl2] = cs[b][r, sl] + cd[b][r, sl2]
                return 0

            lax.fori_loop(0, K, row, 0)
            pltpu.async_copy(pqb, pq_hbm.at[pl.ds(base + j * K, K)], semW)

        fire(0, 0)

        def outer(g, _):
            for bb in (0, 1):
                j = 2 * g + bb

                @pl.when(j + 1 < ch)
                def _():
                    fire(j + 1, 1 - bb)

                @pl.when(j < ch)
                def _():
                    process(j, bb)
            return 0

        lax.fori_loop(0, (ch + 1) // 2, outer, 0)
        pltpu.make_async_copy(pqb, pq_hbm.at[pl.ds(base, K)], semW).wait()

    return body


def _sc_gather(ctab, src, dst):
    f32 = jnp.float32
    cw = ctab.shape[1]
    ne = src.shape[0]
    per_w = ne // NW
    ch = per_w // K
    scratch = [
        pltpu.VMEM((per_w,), jnp.int32),
        pltpu.VMEM((per_w,), jnp.int32),
        pltpu.VMEM((K, cw), f32),
        pltpu.VMEM((K, cw), f32),
        pltpu.VMEM((K, cw), f32),
        pltpu.VMEM((K, cw), f32),
        pltpu.VMEM((K, 2 * H), f32),
    ] + [pltpu.SemaphoreType.DMA] * 5
    k = pl.kernel(
        _make_gather_body(per_w, ch),
        out_type=jax.ShapeDtypeStruct((ne, 2 * H), f32),
        mesh=_sc_mesh,
        scratch_types=scratch,
    )
    return k(ctab, src, dst)


def _make_scatter_body(per_w, ch):
    def body(msg_hbm, dst_hbm, out_hbm, idx0, idx1, mb0, mb1, zbuf,
             aggr_sh, semI0, semI1, semM0, semM1):
        idx = (idx0, idx1)
        mb = (mb0, mb1)
        semI = (semI0, semI1)
        semM = (semM0, semM1)
        c = lax.axis_index("c")
        s = lax.axis_index("s")
        wid = c * NS + s
        base = wid * per_w

        def zrow(r, _):
            for cc in range(H // 16):
                zbuf[r, pl.ds(cc * 16, 16)] = jnp.zeros((16,), jnp.float32)
            return 0

        lax.fori_loop(0, K, zrow, 0)

        def zchunk(j, _):
            @pl.when(lax.rem(j, NS) == s)
            def _():
                pltpu.sync_copy(zbuf, aggr_sh.at[pl.ds(j * K, K)])
            return 0

        lax.fori_loop(0, N // K, zchunk, 0)
        plsc.subcore_barrier()

        def fire(k, b):
            o = base + k * K
            pltpu.async_copy(dst_hbm.at[pl.ds(o, K)], idx[b], semI[b])
            pltpu.async_copy(msg_hbm.at[pl.ds(o, K)], mb[b], semM[b])

        def process(b):
            pltpu.make_async_copy(
                dst_hbm.at[pl.ds(base, K)], idx[b], semI[b]).wait()
            pltpu.make_async_copy(
                msg_hbm.at[pl.ds(base, K)], mb[b], semM[b]).wait()
            pltpu.sync_copy(mb[b], aggr_sh.at[idx[b]], add=True)

        fire(0, 0)

        def outer(g, _):
            for bb in (0, 1):
                j = 2 * g + bb

                @pl.when(j + 1 < ch)
                def _():
                    fire(j + 1, 1 - bb)

                @pl.when(j < ch)
                def _():
                    process(bb)
            return 0

        lax.fori_loop(0, (ch + 1) // 2, outer, 0)
        plsc.subcore_barrier()

        def dchunk(j, _):
            @pl.when(lax.rem(j, NS) == s)
            def _():
                pltpu.sync_copy(aggr_sh.at[pl.ds(j * K, K)],
                                out_hbm.at[c, pl.ds(j * K, K)])
            return 0

        lax.fori_loop(0, N // K, dchunk, 0)

    return body


def _sc_scatter(msg, dst):
    f32 = jnp.float32
    ne = dst.shape[0]
    per_w = ne // NW
    ch = per_w // K
    k = pl.kernel(
        _make_scatter_body(per_w, ch),
        out_type=jax.ShapeDtypeStruct((NC, N, H), f32),
        mesh=_sc_mesh,
        scratch_types=[
            pltpu.VMEM((K,), jnp.int32),
            pltpu.VMEM((K,), jnp.int32),
            pltpu.VMEM((K, H), f32),
            pltpu.VMEM((K, H), f32),
            pltpu.VMEM((K, H), f32),
            pltpu.VMEM_SHARED((N, H), f32),
            pltpu.SemaphoreType.DMA,
            pltpu.SemaphoreType.DMA,
            pltpu.SemaphoreType.DMA,
            pltpu.SemaphoreType.DMA,
        ],
    )
    return k(msg, dst)


def _ln(h, g, b):
    m = jnp.mean(h, axis=-1, keepdims=True)
    v = jnp.mean((h - m) ** 2, axis=-1, keepdims=True)
    return (h - m) * jax.lax.rsqrt(v + 1e-5) * g + b


def _enc_nodes_body(x_ref, w1_ref, b1_ref, w2_ref, b2_ref, g_ref,
                    be_ref, wd_ref, ws_ref, xh_ref, c_ref):
    h = jnp.maximum(
        jnp.dot(x_ref[...], w1_ref[...], preferred_element_type=jnp.float32)
        + b1_ref[...], 0.0)
    xh = _ln(jnp.dot(h, w2_ref[...], preferred_element_type=jnp.float32)
             + b2_ref[...], g_ref[...], be_ref[...])
    xh_ref[...] = xh
    c_ref[:, 0:H] = jnp.dot(xh, wd_ref[...], preferred_element_type=jnp.float32)
    c_ref[:, H:] = jnp.dot(xh, ws_ref[...], preferred_element_type=jnp.float32)


def _enc_nodes(x_feat, p, wd, ws):
    return pl.pallas_call(
        _enc_nodes_body,
        out_shape=[jax.ShapeDtypeStruct((N, H), jnp.float32),
                   jax.ShapeDtypeStruct((N, 2 * H), jnp.float32)],
    )(x_feat, p['W1'], p['b1'], p['W2'], p['b2'], p['g'], p['be'], wd, ws)


def _enc_edges_body(d_ref, w1_ref, b1_ref, w2_ref, b2_ref, g_ref, be_ref,
                    eh_ref):
    d = d_ref[...]
    rel = d[:, 0:3]
    relw = d[:, 3:6]
    dist = jnp.sqrt(jnp.sum(rel * rel, axis=-1, keepdims=True))
    distw = jnp.sqrt(jnp.sum(relw * relw, axis=-1, keepdims=True))
    w1 = w1_ref[...]
    acc = b1_ref[...] + jnp.zeros((d.shape[0], H), jnp.float32)
    for j, col in enumerate((d[:, 0:1], d[:, 1:2], d[:, 2:3], dist,
                             d[:, 3:4], d[:, 4:5], d[:, 5:6], distw,
                             d[:, 6:7])):
        acc = acc + col * w1[j:j + 1, :]
    h = jnp.maximum(acc, 0.0)
    eh_ref[...] = _ln(
        jnp.dot(h, w2_ref[...], preferred_element_type=jnp.float32)
        + b2_ref[...], g_ref[...], be_ref[...])


def _enc_edges(d128, p):
    ne = d128.shape[0]
    wfull = pl.BlockSpec((16, H), lambda i: (0, 0))
    wrow = pl.BlockSpec((1, H), lambda i: (0, 0))
    wsq = pl.BlockSpec((H, H), lambda i: (0, 0))
    return pl.pallas_call(
        _enc_edges_body,
        grid=(ne // BE,),
        in_specs=[pl.BlockSpec((BE, H), lambda i: (i, 0)),
                  wfull, wrow, wsq, wrow, wrow, wrow],
        out_specs=pl.BlockSpec((BE, H), lambda i: (i, 0)),
        out_shape=jax.ShapeDtypeStruct((ne, H), jnp.float32),
    )(d128, p['W1'], p['b1'], p['W2'], p['b2'], p['g'], p['be'])


def _edge_upd_body(eh_ref, pq_ref, we_ref, w2_ref, b1_ref, b2_ref,
                   g_ref, be_ref, msg_ref, eout_ref):
    eh = eh_ref[...]
    ec = jnp.dot(eh, we_ref[...], preferred_element_type=jnp.float32)
    b1 = b1_ref[...]
    g = g_ref[...]
    be = be_ref[...]
    w2 = w2_ref[...]
    b2 = b2_ref[...]
    hm = jnp.maximum(pq_ref[:, 0:H] + ec + b1, 0.0)
    msg_ref[...] = _ln(
        jnp.dot(hm, w2, preferred_element_type=jnp.float32) + b2, g, be)
    hn = jnp.maximum(pq_ref[:, H:] + ec + b1, 0.0)
    ne = _ln(jnp.dot(hn, w2, preferred_element_type=jnp.float32) + b2, g, be)
    eout_ref[...] = eh + ne


def _edge_update(eh, pq, we, w2, b1, b2, g, be):
    ne = eh.shape[0]
    blk = pl.BlockSpec((BE, H), lambda i: (i, 0))
    blk2 = pl.BlockSpec((BE, 2 * H), lambda i: (i, 0))
    wsq = pl.BlockSpec((H, H), lambda i: (0, 0))
    wrow = pl.BlockSpec((1, H), lambda i: (0, 0))
    f = jax.ShapeDtypeStruct((ne, H), jnp.float32)
    return pl.pallas_call(
        _edge_upd_body,
        grid=(ne // BE,),
        in_specs=[blk, blk2, wsq, wsq, wrow, wrow, wrow, wrow],
        out_specs=[blk, blk],
        out_shape=[f, f],
    )(eh, pq, we, w2, b1, b2, g, be)


def _node_upd_body(xh_ref, ag1_ref, ag2_ref, wa_ref, wx_ref, b1_ref, w2_ref,
                   b2_ref, g_ref, be_ref, wd_ref, ws_ref, xo_ref, c_ref):
    xh = xh_ref[...]
    ag = (ag1_ref[0] + ag1_ref[1]) + (ag2_ref[0] + ag2_ref[1])
    h = jnp.maximum(
        jnp.dot(ag, wa_ref[...], preferred_element_type=jnp.float32)
        + jnp.dot(xh, wx_ref[...], preferred_element_type=jnp.float32)
        + b1_ref[...], 0.0)
    nx = _ln(jnp.dot(h, w2_ref[...], preferred_element_type=jnp.float32)
             + b2_ref[...], g_ref[...], be_ref[...])
    xo = xh + nx
    xo_ref[...] = xo
    c_ref[:, 0:H] = jnp.dot(xo, wd_ref[...], preferred_element_type=jnp.float32)
    c_ref[:, H:] = jnp.dot(xo, ws_ref[...], preferred_element_type=jnp.float32)


def _node_update(xh, ag1, ag2, pn, wd, ws):
    return pl.pallas_call(
        _node_upd_body,
        out_shape=[jax.ShapeDtypeStruct((N, H), jnp.float32),
                   jax.ShapeDtypeStruct((N, 2 * H), jnp.float32)],
    )(xh, ag1, ag2, pn['W1'][:H], pn['W1'][H:], pn['b1'], pn['W2'], pn['b2'],
      pn['g'], pn['be'], wd, ws)


def _node_dec_body(xh_ref, ag1_ref, ag2_ref, wa_ref, wx_ref, b1_ref, w2_ref,
                   b2_ref, g_ref, be_ref, dw1_ref, db1_ref, dw2_ref, db2_ref,
                   y_ref):
    xh = xh_ref[...]
    ag = (ag1_ref[0] + ag1_ref[1]) + (ag2_ref[0] + ag2_ref[1])
    h = jnp.maximum(
        jnp.dot(ag, wa_ref[...], preferred_element_type=jnp.float32)
        + jnp.dot(xh, wx_ref[...], preferred_element_type=jnp.float32)
        + b1_ref[...], 0.0)
    nx = _ln(jnp.dot(h, w2_ref[...], preferred_element_type=jnp.float32)
             + b2_ref[...], g_ref[...], be_ref[...])
    xo = xh + nx
    dh = jnp.maximum(
        jnp.dot(xo, dw1_ref[...], preferred_element_type=jnp.float32)
        + db1_ref[...], 0.0)
    y_ref[...] = jnp.dot(dh, dw2_ref[...], preferred_element_type=jnp.float32) \
        + db2_ref[...]


def _node_dec(xh, ag1, ag2, pn, dec, dw2pad, db2pad):
    return pl.pallas_call(
        _node_dec_body,
        out_shape=jax.ShapeDtypeStruct((N, H), jnp.float32),
    )(xh, ag1, ag2, pn['W1'][:H], pn['W1'][H:], pn['b1'], pn['W2'], pn['b2'],
      pn['g'], pn['be'], dec['W1'], dec['b1'], dw2pad, db2pad)


def kernel(world_pos, mesh_pos, phi, swelling_phi, next_swelling_phi,
           rate_swelling_phi, node_type, mat_param, edge_index, params):
    src = edge_index[0]
    dst = edge_index[1]

    u = world_pos - mesh_pos
    mat = jnp.broadcast_to(mat_param[None, :], (N, 4))
    x_feat = jnp.concatenate(
        [u, phi, swelling_phi, next_swelling_phi, rate_swelling_phi,
         node_type, mat], axis=-1)
    x_feat = jnp.pad(x_feat, ((0, 0), (0, 12)))

    pe_n = params['node_enc']
    ne_p = {'W1': jnp.pad(pe_n['W1'], ((0, 12), (0, 0))),
            'b1': pe_n['b1'][None, :], 'W2': pe_n['W2'],
            'b2': pe_n['b2'][None, :], 'g': pe_n['g'][None, :],
            'be': pe_n['be'][None, :]}

    procs = params['procs']
    wd0 = procs[0]['edge']['W1'][:H]
    ws0 = procs[0]['edge']['W1'][H:2 * H]

    g128 = jnp.pad(jnp.concatenate([mesh_pos, world_pos, phi], axis=-1),
                   ((0, 0), (0, H - 7)))

    x_h, C = _enc_nodes(x_feat, ne_p, wd0, ws0)

    E2 = E // 2
    s1, s2 = src[:E2], src[E2:]
    d1, d2 = dst[:E2], dst[E2:]

    pe_e = params['edge_enc']
    ee_p = {'W1': jnp.pad(pe_e['W1'], ((0, 7), (0, 0))),
            'b1': pe_e['b1'][None, :], 'W2': pe_e['W2'],
            'b2': pe_e['b2'][None, :], 'g': pe_e['g'][None, :],
            'be': pe_e['be'][None, :]}

    dg1 = _sc_geom(g128, s1, d1)
    dg2 = _sc_geom(g128, s2, d2)
    eh1 = _enc_edges(dg1, ee_p)
    eh2 = _enc_edges(dg2, ee_p)

    for l in range(len(procs)):
        pe = procs[l]['edge']
        pn = procs[l]['node']
        we = pe['W1'][2 * H:]
        eb1 = pe['b1'][None, :]
        eb2 = pe['b2'][None, :]
        eg = pe['g'][None, :]
        ebe = pe['be'][None, :]
        pq1 = _sc_gather(C, s1, d1)
        pq2 = _sc_gather(C, s2, d2)
        msg1, eh1 = _edge_update(eh1, pq1, we, pe['W2'], eb1, eb2, eg, ebe)
        ag1 = _sc_scatter(msg1, d1)
        msg2, eh2 = _edge_update(eh2, pq2, we, pe['W2'], eb1, eb2, eg, ebe)
        ag2 = _sc_scatter(msg2, d2)
        pn_p = {'W1': pn['W1'], 'b1': pn['b1'][None, :], 'W2': pn['W2'],
                'b2': pn['b2'][None, :], 'g': pn['g'][None, :],
                'be': pn['be'][None, :]}
        if l + 1 < len(procs):
            wd = procs[l + 1]['edge']['W1'][:H]
            ws = procs[l + 1]['edge']['W1'][H:2 * H]
            x_h, C = _node_update(x_h, ag1, ag2, pn_p, wd, ws)
        else:
            dec = params['dec']
            dec_p = {'W1': dec['W1'], 'b1': dec['b1'][None, :]}
            dw2pad = jnp.pad(dec['W2'], ((0, 0), (0, H - 3)))
            db2pad = jnp.pad(dec['b2'], (0, H - 3))[None, :]
            y = _node_dec(x_h, ag1, ag2, pn_p, dec_p, dw2pad, db2pad)
    return y[:, :3]


# speedup vs baseline: 1.0268x; 1.0268x over previous
import functools

import jax
import jax.numpy as jnp
from jax import lax
from jax.experimental import pallas as pl
from jax.experimental.pallas import tpu as pltpu
from jax.experimental.pallas import tpu_sc as plsc

N = 10000
E = 320000
H = 128
BE = 4000

NC = 2
NS = 16
NW = NC * NS
PER_W = E // NW
K = 40

_sc_mesh = plsc.VectorSubcoreMesh(core_axis_name="c", subcore_axis_name="s")


def _make_geom_body(per_w, ch):
    def body(g_hbm, src_hbm, dst_hbm, d_hbm,
             srcv, dstv, gs0, gs1, gd0, gd1, dbuf,
             semS0, semS1, semD0, semD1, semW):
        gs = (gs0, gs1)
        gd = (gd0, gd1)
        semS = (semS0, semS1)
        semD = (semD0, semD1)
        c = lax.axis_index("c")
        s = lax.axis_index("s")
        wid = c * NS + s
        base = wid * per_w
        pltpu.sync_copy(src_hbm.at[pl.ds(base, per_w)], srcv)
        pltpu.sync_copy(dst_hbm.at[pl.ds(base, per_w)], dstv)

        def fire(k, b):
            o = k * K
            pltpu.async_copy(g_hbm.at[srcv.at[pl.ds(o, K)]], gs[b], semS[b])
            pltpu.async_copy(g_hbm.at[dstv.at[pl.ds(o, K)]], gd[b], semD[b])

        def process(j, b):
            pltpu.make_async_copy(
                g_hbm.at[srcv.at[pl.ds(0, K)]], gs[b], semS[b]).wait()
            pltpu.make_async_copy(
                g_hbm.at[dstv.at[pl.ds(0, K)]], gd[b], semD[b]).wait()

            @pl.when(j > 0)
            def _():
                pltpu.make_async_copy(
                    dbuf, d_hbm.at[pl.ds(base, K)], semW).wait()

            def row(r, _):
                sl = pl.ds(0, 16)
                dbuf[r, sl] = gs[b][r, sl] - gd[b][r, sl]
                return 0

            lax.fori_loop(0, K, row, 0)
            pltpu.async_copy(dbuf, d_hbm.at[pl.ds(base + j * K, K)], semW)

        fire(0, 0)

        def outer(g, _):
            for bb in (0, 1):
                j = 2 * g + bb

                @pl.when(j + 1 < ch)
                def _():
                    fire(j + 1, 1 - bb)

                @pl.when(j < ch)
                def _():
                    process(j, bb)
            return 0

        lax.fori_loop(0, (ch + 1) // 2, outer, 0)
        pltpu.make_async_copy(dbuf, d_hbm.at[pl.ds(base, K)], semW).wait()

    return body


def _sc_geom(g128, src, dst):
    f32 = jnp.float32
    ne = src.shape[0]
    per_w = ne // NW
    ch = per_w // K
    k = pl.kernel(
        _make_geom_body(per_w, ch),
        out_type=jax.ShapeDtypeStruct((ne, H), f32),
        mesh=_sc_mesh,
        scratch_types=[
            pltpu.VMEM((per_w,), jnp.int32),
            pltpu.VMEM((per_w,), jnp.int32),
            pltpu.VMEM((K, H), f32),
            pltpu.VMEM((K, H), f32),
            pltpu.VMEM((K, H), f32),
            pltpu.VMEM((K, H), f32),
            pltpu.VMEM((K, H), f32),
        ] + [pltpu.SemaphoreType.DMA] * 5,
    )
    return k(g128, src, dst)


def _make_gather_body(per_w, ch):
    def body(c_hbm, src_hbm, dst_hbm, pq_hbm, srcv, dstv,
             cs0, cs1, cd0, cd1, pqb, semS0, semS1, semD0, semD1, semW):
        cs = (cs0, cs1)
        cd = (cd0, cd1)
        semS = (semS0, semS1)
        semD = (semD0, semD1)
        c = lax.axis_index("c")
        s = lax.axis_index("s")
        wid = c * NS + s
        base = wid * per_w
        pltpu.sync_copy(src_hbm.at[pl.ds(base, per_w)], srcv)
        pltpu.sync_copy(dst_hbm.at[pl.ds(base, per_w)], dstv)

        def fire(k, b):
            o = k * K
            pltpu.async_copy(c_hbm.at[srcv.at[pl.ds(o, K)]], cs[b], semS[b])
            pltpu.async_copy(c_hbm.at[dstv.at[pl.ds(o, K)]], cd[b], semD[b])

        def process(j, b):
            pltpu.make_async_copy(
                c_hbm.at[srcv.at[pl.ds(0, K)]], cs[b], semS[b]).wait()
            pltpu.make_async_copy(
                c_hbm.at[dstv.at[pl.ds(0, K)]], cd[b], semD[b]).wait()

            @pl.when(j > 0)
            def _():
                pltpu.make_async_copy(
                    pqb, pq_hbm.at[pl.ds(base, K)], semW).wait()

            def row(r, _):
                for cc in range(H // 16):
                    sl = pl.ds(cc * 16, 16)
                    sl2 = pl.ds(H + cc * 16, 16)
                    pqb[r, sl] = cd[b][r, sl] + cs[b][r, sl2]
                    pqb[r, sl2] = cs[b][r, sl] + cd[b][r, sl2]
                return 0

            lax.fori_loop(0, K, row, 0)
            pltpu.async_copy(pqb, pq_hbm.at[pl.ds(base + j * K, K)], semW)

        fire(0, 0)

        def outer(g, _):
            for bb in (0, 1):
                j = 2 * g + bb

                @pl.when(j + 1 < ch)
                def _():
                    fire(j + 1, 1 - bb)

                @pl.when(j < ch)
                def _():
                    process(j, bb)
            return 0

        lax.fori_loop(0, (ch + 1) // 2, outer, 0)
        pltpu.make_async_copy(pqb, pq_hbm.at[pl.ds(base, K)], semW).wait()

    return body


def _sc_gather(ctab, src, dst):
    f32 = jnp.float32
    cw = ctab.shape[1]
    ne = src.shape[0]
    per_w = ne // NW
    ch = per_w // K
    scratch = [
        pltpu.VMEM((per_w,), jnp.int32),
        pltpu.VMEM((per_w,), jnp.int32),
        pltpu.VMEM((K, cw), f32),
        pltpu.VMEM((K, cw), f32),
        pltpu.VMEM((K, cw), f32),
        pltpu.VMEM((K, cw), f32),
        pltpu.VMEM((K, 2 * H), f32),
    ] + [pltpu.SemaphoreType.DMA] * 5
    k = pl.kernel(
        _make_gather_body(per_w, ch),
        out_type=jax.ShapeDtypeStruct((ne, 2 * H), f32),
        mesh=_sc_mesh,
        scratch_types=scratch,
    )
    return k(ctab, src, dst)


def _make_scatter_body(per_w, ch):
    def body(msg_hbm, dst_hbm, out_hbm, idx0, idx1, mb0, mb1, zbuf,
             aggr_sh, semI0, semI1, semM0, semM1):
        idx = (idx0, idx1)
        mb = (mb0, mb1)
        semI = (semI0, semI1)
        semM = (semM0, semM1)
        c = lax.axis_index("c")
        s = lax.axis_index("s")
        wid = c * NS + s
        base = wid * per_w

        def zrow(r, _):
            for cc in range(H // 16):
                zbuf[r, pl.ds(cc * 16, 16)] = jnp.zeros((16,), jnp.float32)
            return 0

        lax.fori_loop(0, K, zrow, 0)

        def zchunk(j, _):
            @pl.when(lax.rem(j, NS) == s)
            def _():
                pltpu.sync_copy(zbuf, aggr_sh.at[pl.ds(j * K, K)])
            return 0

        lax.fori_loop(0, N // K, zchunk, 0)
        plsc.subcore_barrier()

        def fire(k, b):
            o = base + k * K
            pltpu.async_copy(dst_hbm.at[pl.ds(o, K)], idx[b], semI[b])
            pltpu.async_copy(msg_hbm.at[pl.ds(o, K)], mb[b], semM[b])

        def process(b):
            pltpu.make_async_copy(
                dst_hbm.at[pl.ds(base, K)], idx[b], semI[b]).wait()
            pltpu.make_async_copy(
                msg_hbm.at[pl.ds(base, K)], mb[b], semM[b]).wait()
            pltpu.sync_copy(mb[b], aggr_sh.at[idx[b]], add=True)

        fire(0, 0)

        def outer(g, _):
            for bb in (0, 1):
                j = 2 * g + bb

                @pl.when(j + 1 < ch)
                def _():
                    fire(j + 1, 1 - bb)

                @pl.when(j < ch)
                def _():
                    process(bb)
            return 0

        lax.fori_loop(0, (ch + 1) // 2, outer, 0)
        plsc.subcore_barrier()

        def dchunk(j, _):
            @pl.when(lax.rem(j, NS) == s)
            def _():
                pltpu.sync_copy(aggr_sh.at[pl.ds(j * K, K)],
                                out_hbm.at[c, pl.ds(j * K, K)])
            return 0

        lax.fori_loop(0, N // K, dchunk, 0)

    return body


def _sc_scatter(msg, dst):
    f32 = jnp.float32
    ne = dst.shape[0]
    per_w = ne // NW
    ch = per_w // K
    k = pl.kernel(
        _make_scatter_body(per_w, ch),
        out_type=jax.ShapeDtypeStruct((NC, N, H), f32),
        mesh=_sc_mesh,
        scratch_types=[
            pltpu.VMEM((K,), jnp.int32),
            pltpu.VMEM((K,), jnp.int32),
            pltpu.VMEM((K, H), f32),
            pltpu.VMEM((K, H), f32),
            pltpu.VMEM((K, H), f32),
            pltpu.VMEM_SHARED((N, H), f32),
            pltpu.SemaphoreType.DMA,
            pltpu.SemaphoreType.DMA,
            pltpu.SemaphoreType.DMA,
            pltpu.SemaphoreType.DMA,
        ],
    )
    return k(msg, dst)


def _ln(h, g, b):
    m = jnp.mean(h, axis=-1, keepdims=True)
    v = jnp.mean((h - m) ** 2, axis=-1, keepdims=True)
    return (h - m) * jax.lax.rsqrt(v + 1e-5) * g + b


def _enc_nodes_body(x_ref, w1_ref, b1_ref, w2_ref, b2_ref, g_ref,
                    be_ref, wd_ref, ws_ref, xh_ref, c_ref):
    h = jnp.maximum(
        jnp.dot(x_ref[...], w1_ref[...], preferred_element_type=jnp.float32)
        + b1_ref[...], 0.0)
    xh = _ln(jnp.dot(h, w2_ref[...], preferred_element_type=jnp.float32)
             + b2_ref[...], g_ref[...], be_ref[...])
    xh_ref[...] = xh
    c_ref[:, 0:H] = jnp.dot(xh, wd_ref[...], preferred_element_type=jnp.float32)
    c_ref[:, H:] = jnp.dot(xh, ws_ref[...], preferred_element_type=jnp.float32)


def _enc_nodes(x_feat, p, wd, ws):
    return pl.pallas_call(
        _enc_nodes_body,
        out_shape=[jax.ShapeDtypeStruct((N, H), jnp.float32),
                   jax.ShapeDtypeStruct((N, 2 * H), jnp.float32)],
    )(x_feat, p['W1'], p['b1'], p['W2'], p['b2'], p['g'], p['be'], wd, ws)


def _enc_edges_body(d_ref, w1_ref, b1_ref, w2_ref, b2_ref, g_ref, be_ref,
                    eh_ref):
    d = d_ref[...]
    rel = d[:, 0:3]
    relw = d[:, 3:6]
    dist = jnp.sqrt(jnp.sum(rel * rel, axis=-1, keepdims=True))
    distw = jnp.sqrt(jnp.sum(relw * relw, axis=-1, keepdims=True))
    w1 = w1_ref[...]
    acc = b1_ref[...] + jnp.zeros((d.shape[0], H), jnp.float32)
    for j, col in enumerate((d[:, 0:1], d[:, 1:2], d[:, 2:3], dist,
                             d[:, 3:4], d[:, 4:5], d[:, 5:6], distw,
                             d[:, 6:7])):
        acc = acc + col * w1[j:j + 1, :]
    h = jnp.maximum(acc, 0.0)
    eh_ref[...] = _ln(
        jnp.dot(h, w2_ref[...], preferred_element_type=jnp.float32)
        + b2_ref[...], g_ref[...], be_ref[...])


def _enc_edges(d128, p):
    ne = d128.shape[0]
    wfull = pl.BlockSpec((16, H), lambda i: (0, 0))
    wrow = pl.BlockSpec((1, H), lambda i: (0, 0))
    wsq = pl.BlockSpec((H, H), lambda i: (0, 0))
    return pl.pallas_call(
        _enc_edges_body,
        grid=(ne // BE,),
        in_specs=[pl.BlockSpec((BE, H), lambda i: (i, 0)),
                  wfull, wrow, wsq, wrow, wrow, wrow],
        out_specs=pl.BlockSpec((BE, H), lambda i: (i, 0)),
        out_shape=jax.ShapeDtypeStruct((ne, H), jnp.float32),
    )(d128, p['W1'], p['b1'], p['W2'], p['b2'], p['g'], p['be'])


def _edge_upd_body(eh_ref, pq_ref, we_ref, w2_ref, b1_ref, b2_ref,
                   g_ref, be_ref, msg_ref, eout_ref):
    eh = eh_ref[...]
    ec = jnp.dot(eh, we_ref[...], preferred_element_type=jnp.float32)
    b1 = b1_ref[...]
    g = g_ref[...]
    be = be_ref[...]
    w2 = w2_ref[...]
    b2 = b2_ref[...]
    hm = jnp.maximum(pq_ref[:, 0:H] + ec + b1, 0.0)
    msg_ref[...] = _ln(
        jnp.dot(hm, w2, preferred_element_type=jnp.float32) + b2, g, be)
    hn = jnp.maximum(pq_ref[:, H:] + ec + b1, 0.0)
    ne = _ln(jnp.dot(hn, w2, preferred_element_type=jnp.float32) + b2, g, be)
    eout_ref[...] = eh + ne


def _edge_update(eh, pq, we, w2, b1, b2, g, be):
    ne = eh.shape[0]
    blk = pl.BlockSpec((BE, H), lambda i: (i, 0))
    blk2 = pl.BlockSpec((BE, 2 * H), lambda i: (i, 0))
    wsq = pl.BlockSpec((H, H), lambda i: (0, 0))
    wrow = pl.BlockSpec((1, H), lambda i: (0, 0))
    f = jax.ShapeDtypeStruct((ne, H), jnp.float32)
    return pl.pallas_call(
        _edge_upd_body,
        grid=(ne // BE,),
        in_specs=[blk, blk2, wsq, wsq, wrow, wrow, wrow, wrow],
        out_specs=[blk, blk],
        out_shape=[f, f],
    )(eh, pq, we, w2, b1, b2, g, be)


def _node_upd_body(xh_ref, ag1_ref, ag2_ref, wa_ref, wx_ref, b1_ref, w2_ref,
                   b2_ref, g_ref, be_ref, wd_ref, ws_ref, xo_ref, c_ref):
    xh = xh_ref[...]
    ag = (ag1_ref[0] + ag1_ref[1]) + (ag2_ref[0] + ag2_ref[1])
    h = jnp.maximum(
        jnp.dot(ag, wa_ref[...], preferred_element_type=jnp.float32)
        + jnp.dot(xh, wx_ref[...], preferred_element_type=jnp.float32)
        + b1_ref[...], 0.0)
    nx = _ln(jnp.dot(h, w2_ref[...], preferred_element_type=jnp.float32)
             + b2_ref[...], g_ref[...], be_ref[...])
    xo = xh + nx
    xo_ref[...] = xo
    c_ref[:, 0:H] = jnp.dot(xo, wd_ref[...], preferred_element_type=jnp.float32)
    c_ref[:, H:] = jnp.dot(xo, ws_ref[...], preferred_element_type=jnp.float32)


def _node_update(xh, ag1, ag2, pn, wd, ws):
    return pl.pallas_call(
        _node_upd_body,
        out_shape=[jax.ShapeDtypeStruct((N, H), jnp.float32),
                   jax.ShapeDtypeStruct((N, 2 * H), jnp.float32)],
    )(xh, ag1, ag2, pn['W1'][:H], pn['W1'][H:], pn['b1'], pn['W2'], pn['b2'],
      pn['g'], pn['be'], wd, ws)


def _node_dec_body(xh_ref, ag1_ref, ag2_ref, wa_ref, wx_ref, b1_ref, w2_ref,
                   b2_ref, g_ref, be_ref, dw1_ref, db1_ref, dw2_ref, db2_ref,
                   y_ref):
    xh = xh_ref[...]
    ag = (ag1_ref[0] + ag1_ref[1]) + (ag2_ref[0] + ag2_ref[1])
    h = jnp.maximum(
        jnp.dot(ag, wa_ref[...], preferred_element_type=jnp.float32)
        + jnp.dot(xh, wx_ref[...], preferred_element_type=jnp.float32)
        + b1_ref[...], 0.0)
    nx = _ln(jnp.dot(h, w2_ref[...], preferred_element_type=jnp.float32)
             + b2_ref[...], g_ref[...], be_ref[...])
    xo = xh + nx
    dh = jnp.maximum(
        jnp.dot(xo, dw1_ref[...], preferred_element_type=jnp.float32)
        + db1_ref[...], 0.0)
    y_ref[...] = jnp.dot(dh, dw2_ref[...], preferred_element_type=jnp.float32) \
        + db2_ref[...]


def _node_dec(xh, ag1, ag2, pn, dec, dw2pad, db2pad):
    return pl.pallas_call(
        _node_dec_body,
        out_shape=jax.ShapeDtypeStruct((N, H), jnp.float32),
    )(xh, ag1, ag2, pn['W1'][:H], pn['W1'][H:], pn['b1'], pn['W2'], pn['b2'],
      pn['g'], pn['be'], dec['W1'], dec['b1'], dw2pad, db2pad)


def kernel(world_pos, mesh_pos, phi, swelling_phi, next_swelling_phi,
           rate_swelling_phi, node_type, mat_param, edge_index, params):
    src = edge_index[0]
    dst = edge_index[1]

    u = world_pos - mesh_pos
    mat = jnp.broadcast_to(mat_param[None, :], (N, 4))
    x_feat = jnp.concatenate(
        [u, phi, swelling_phi, next_swelling_phi, rate_swelling_phi,
         node_type, mat], axis=-1)
    x_feat = jnp.pad(x_feat, ((0, 0), (0, 12)))

    pe_n = params['node_enc']
    ne_p = {'W1': jnp.pad(pe_n['W1'], ((0, 12), (0, 0))),
            'b1': pe_n['b1'][None, :], 'W2': pe_n['W2'],
            'b2': pe_n['b2'][None, :], 'g': pe_n['g'][None, :],
            'be': pe_n['be'][None, :]}

    procs = params['procs']
    wd0 = procs[0]['edge']['W1'][:H]
    ws0 = procs[0]['edge']['W1'][H:2 * H]

    g128 = jnp.pad(jnp.concatenate([mesh_pos, world_pos, phi], axis=-1),
                   ((0, 0), (0, H - 7)))

    x_h, C = _enc_nodes(x_feat, ne_p, wd0, ws0)

    E2 = E // 2
    s1, s2 = src[:E2], src[E2:]
    d1, d2 = dst[:E2], dst[E2:]

    pe_e = params['edge_enc']
    ee_p = {'W1': jnp.pad(pe_e['W1'], ((0, 7), (0, 0))),
            'b1': pe_e['b1'][None, :], 'W2': pe_e['W2'],
            'b2': pe_e['b2'][None, :], 'g': pe_e['g'][None, :],
            'be': pe_e['be'][None, :]}

    dg1 = _sc_geom(g128, s1, d1)
    dg2 = _sc_geom(g128, s2, d2)
    eh1 = _enc_edges(dg1, ee_p)
    eh2 = _enc_edges(dg2, ee_p)

    for l in range(len(procs)):
        pe = procs[l]['edge']
        pn = procs[l]['node']
        we = pe['W1'][2 * H:]
        eb1 = pe['b1'][None, :]
        eb2 = pe['b2'][None, :]
        eg = pe['g'][None, :]
        ebe = pe['be'][None, :]
        pq1 = _sc_gather(C, s1, d1)
        pq2 = _sc_gather(C, s2, d2)
        msg1, eh1 = _edge_update(eh1, pq1, we, pe['W2'], eb1, eb2, eg, ebe)
        ag1 = _sc_scatter(msg1, d1)
        msg2, eh2 = _edge_update(eh2, pq2, we, pe['W2'], eb1, eb2, eg, ebe)
        ag2 = _sc_scatter(msg2, d2)
        pn_p = {'W1': pn['W1'], 'b1': pn['b1'][None, :], 'W2': pn['W2'],
                'b2': pn['b2'][None, :], 'g': pn['g'][None, :],
                'be': pn['be'][None, :]}
        if l + 1 < len(procs):
            wd = procs[l + 1]['edge']['W1'][:H]
            ws = procs[l + 1]['edge']['W1'][H:2 * H]
            x_h, C = _node_update(x_h, ag1, ag2, pn_p, wd, ws)
        else:
            dec = params['dec']
            dec_p = {'W1': dec['W1'], 'b1': dec['b1'][None, :]}
            dw2pad = jnp.pad(dec['W2'], ((0, 0), (0, H - 3)))
            db2pad = jnp.pad(dec['b2'], (0, H - 3))[None, :]
            y = _node_dec(x_h, ag1, ag2, pn_p, dec_p, dw2pad, db2pad)
    return y[:, :3]


# speedup vs baseline: 1.0311x; 1.0043x over previous
import functools

import jax
import jax.numpy as jnp
from jax import lax
from jax.experimental import pallas as pl
from jax.experimental.pallas import tpu as pltpu
from jax.experimental.pallas import tpu_sc as plsc

N = 10000
E = 320000
H = 128
BE = 8000

NC = 2
NS = 16
NW = NC * NS
PER_W = E // NW
K = 40

_sc_mesh = plsc.VectorSubcoreMesh(core_axis_name="c", subcore_axis_name="s")


def _make_geom_body(per_w, ch):
    def body(g_hbm, src_hbm, dst_hbm, d_hbm,
             srcv, dstv, gs0, gs1, gd0, gd1, dbuf,
             semS0, semS1, semD0, semD1, semW):
        gs = (gs0, gs1)
        gd = (gd0, gd1)
        semS = (semS0, semS1)
        semD = (semD0, semD1)
        c = lax.axis_index("c")
        s = lax.axis_index("s")
        wid = c * NS + s
        base = wid * per_w
        pltpu.sync_copy(src_hbm.at[pl.ds(base, per_w)], srcv)
        pltpu.sync_copy(dst_hbm.at[pl.ds(base, per_w)], dstv)

        def fire(k, b):
            o = k * K
            pltpu.async_copy(g_hbm.at[srcv.at[pl.ds(o, K)]], gs[b], semS[b])
            pltpu.async_copy(g_hbm.at[dstv.at[pl.ds(o, K)]], gd[b], semD[b])

        def process(j, b):
            pltpu.make_async_copy(
                g_hbm.at[srcv.at[pl.ds(0, K)]], gs[b], semS[b]).wait()
            pltpu.make_async_copy(
                g_hbm.at[dstv.at[pl.ds(0, K)]], gd[b], semD[b]).wait()

            @pl.when(j > 0)
            def _():
                pltpu.make_async_copy(
                    dbuf, d_hbm.at[pl.ds(base, K)], semW).wait()

            def row(r, _):
                sl = pl.ds(0, 16)
                dbuf[r, sl] = gs[b][r, sl] - gd[b][r, sl]
                return 0

            lax.fori_loop(0, K, row, 0)
            pltpu.async_copy(dbuf, d_hbm.at[pl.ds(base + j * K, K)], semW)

        fire(0, 0)

        def outer(g, _):
            for bb in (0, 1):
                j = 2 * g + bb

                @pl.when(j + 1 < ch)
                def _():
                    fire(j + 1, 1 - bb)

                @pl.when(j < ch)
                def _():
                    process(j, bb)
            return 0

        lax.fori_loop(0, (ch + 1) // 2, outer, 0)
        pltpu.make_async_copy(dbuf, d_hbm.at[pl.ds(base, K)], semW).wait()

    return body


def _sc_geom(g128, src, dst):
    f32 = jnp.float32
    ne = src.shape[0]
    per_w = ne // NW
    ch = per_w // K
    k = pl.kernel(
        _make_geom_body(per_w, ch),
        out_type=jax.ShapeDtypeStruct((ne, H), f32),
        mesh=_sc_mesh,
        scratch_types=[
            pltpu.VMEM((per_w,), jnp.int32),
            pltpu.VMEM((per_w,), jnp.int32),
            pltpu.VMEM((K, H), f32),
            pltpu.VMEM((K, H), f32),
            pltpu.VMEM((K, H), f32),
            pltpu.VMEM((K, H), f32),
            pltpu.VMEM((K, H), f32),
        ] + [pltpu.SemaphoreType.DMA] * 5,
    )
    return k(g128, src, dst)


def _make_gather_body(per_w, ch):
    def body(c_hbm, src_hbm, dst_hbm, pq_hbm, srcv, dstv,
             cs0, cs1, cd0, cd1, pqb, semS0, semS1, semD0, semD1, semW):
        cs = (cs0, cs1)
        cd = (cd0, cd1)
        semS = (semS0, semS1)
        semD = (semD0, semD1)
        c = lax.axis_index("c")
        s = lax.axis_index("s")
        wid = c * NS + s
        base = wid * per_w
        pltpu.sync_copy(src_hbm.at[pl.ds(base, per_w)], srcv)
        pltpu.sync_copy(dst_hbm.at[pl.ds(base, per_w)], dstv)

        def fire(k, b):
            o = k * K
            pltpu.async_copy(c_hbm.at[srcv.at[pl.ds(o, K)]], cs[b], semS[b])
            pltpu.async_copy(c_hbm.at[dstv.at[pl.ds(o, K)]], cd[b], semD[b])

        def process(j, b):
            pltpu.make_async_copy(
                c_hbm.at[srcv.at[pl.ds(0, K)]], cs[b], semS[b]).wait()
            pltpu.make_async_copy(
                c_hbm.at[dstv.at[pl.ds(0, K)]], cd[b], semD[b]).wait()

            @pl.when(j > 0)
            def _():
                pltpu.make_async_copy(
                    pqb, pq_hbm.at[pl.ds(base, K)], semW).wait()

            def row(r, _):
                for cc in range(H // 16):
                    sl = pl.ds(cc * 16, 16)
                    sl2 = pl.ds(H + cc * 16, 16)
                    pqb[r, sl] = cd[b][r, sl] + cs[b][r, sl2]
                    pqb[r, sl2] = cs[b][r, sl] + cd[b][r, sl2]
                return 0

            lax.fori_loop(0, K, row, 0)
            pltpu.async_copy(pqb, pq_hbm.at[pl.ds(base + j * K, K)], semW)

        fire(0, 0)

        def outer(g, _):
            for bb in (0, 1):
                j = 2 * g + bb

                @pl.when(j + 1 < ch)
                def _():
                    fire(j + 1, 1 - bb)

                @pl.when(j < ch)
                def _():
                    process(j, bb)
            return 0

        lax.fori_loop(0, (ch + 1) // 2, outer, 0)
        pltpu.make_async_copy(pqb, pq_hbm.at[pl.ds(base, K)], semW).wait()

    return body


def _sc_gather(ctab, src, dst):
    f32 = jnp.float32
    cw = ctab.shape[1]
    ne = src.shape[0]
    per_w = ne // NW
    ch = per_w // K
    scratch = [
        pltpu.VMEM((per_w,), jnp.int32),
        pltpu.VMEM((per_w,), jnp.int32),
        pltpu.VMEM((K, cw), f32),
        pltpu.VMEM((K, cw), f32),
        pltpu.VMEM((K, cw), f32),
        pltpu.VMEM((K, cw), f32),
        pltpu.VMEM((K, 2 * H), f32),
    ] + [pltpu.SemaphoreType.DMA] * 5
    k = pl.kernel(
        _make_gather_body(per_w, ch),
        out_type=jax.ShapeDtypeStruct((ne, 2 * H), f32),
        mesh=_sc_mesh,
        scratch_types=scratch,
    )
    return k(ctab, src, dst)


def _make_scatter_body(per_w, ch):
    def body(msg_hbm, dst_hbm, out_hbm, idx0, idx1, mb0, mb1, zbuf,
             aggr_sh, semI0, semI1, semM0, semM1):
        idx = (idx0, idx1)
        mb = (mb0, mb1)
        semI = (semI0, semI1)
        semM = (semM0, semM1)
        c = lax.axis_index("c")
        s = lax.axis_index("s")
        wid = c * NS + s
        base = wid * per_w

        def zrow(r, _):
            for cc in range(H // 16):
                zbuf[r, pl.ds(cc * 16, 16)] = jnp.zeros((16,), jnp.float32)
            return 0

        lax.fori_loop(0, K, zrow, 0)

        def zchunk(j, _):
            @pl.when(lax.rem(j, NS) == s)
            def _():
                pltpu.sync_copy(zbuf, aggr_sh.at[pl.ds(j * K, K)])
            return 0

        lax.fori_loop(0, N // K, zchunk, 0)
        plsc.subcore_barrier()

        def fire(k, b):
            o = base + k * K
            pltpu.async_copy(dst_hbm.at[pl.ds(o, K)], idx[b], semI[b])
            pltpu.async_copy(msg_hbm.at[pl.ds(o, K)], mb[b], semM[b])

        def process(b):
            pltpu.make_async_copy(
                dst_hbm.at[pl.ds(base, K)], idx[b], semI[b]).wait()
            pltpu.make_async_copy(
                msg_hbm.at[pl.ds(base, K)], mb[b], semM[b]).wait()
            pltpu.sync_copy(mb[b], aggr_sh.at[idx[b]], add=True)

        fire(0, 0)

        def outer(g, _):
            for bb in (0, 1):
                j = 2 * g + bb

                @pl.when(j + 1 < ch)
                def _():
                    fire(j + 1, 1 - bb)

                @pl.when(j < ch)
                def _():
                    process(bb)
            return 0

        lax.fori_loop(0, (ch + 1) // 2, outer, 0)
        plsc.subcore_barrier()

        def dchunk(j, _):
            @pl.when(lax.rem(j, NS) == s)
            def _():
                pltpu.sync_copy(aggr_sh.at[pl.ds(j * K, K)],
                                out_hbm.at[c, pl.ds(j * K, K)])
            return 0

        lax.fori_loop(0, N // K, dchunk, 0)

    return body


def _sc_scatter(msg, dst):
    f32 = jnp.float32
    ne = dst.shape[0]
    per_w = ne // NW
    ch = per_w // K
    k = pl.kernel(
        _make_scatter_body(per_w, ch),
        out_type=jax.ShapeDtypeStruct((NC, N, H), f32),
        mesh=_sc_mesh,
        scratch_types=[
            pltpu.VMEM((K,), jnp.int32),
            pltpu.VMEM((K,), jnp.int32),
            pltpu.VMEM((K, H), f32),
            pltpu.VMEM((K, H), f32),
            pltpu.VMEM((K, H), f32),
            pltpu.VMEM_SHARED((N, H), f32),
            pltpu.SemaphoreType.DMA,
            pltpu.SemaphoreType.DMA,
            pltpu.SemaphoreType.DMA,
            pltpu.SemaphoreType.DMA,
        ],
    )
    return k(msg, dst)


def _ln(h, g, b):
    m = jnp.mean(h, axis=-1, keepdims=True)
    v = jnp.mean((h - m) ** 2, axis=-1, keepdims=True)
    return (h - m) * jax.lax.rsqrt(v + 1e-5) * g + b


def _enc_nodes_body(x_ref, w1_ref, b1_ref, w2_ref, b2_ref, g_ref,
                    be_ref, wd_ref, ws_ref, xh_ref, c_ref):
    h = jnp.maximum(
        jnp.dot(x_ref[...], w1_ref[...], preferred_element_type=jnp.float32)
        + b1_ref[...], 0.0)
    xh = _ln(jnp.dot(h, w2_ref[...], preferred_element_type=jnp.float32)
             + b2_ref[...], g_ref[...], be_ref[...])
    xh_ref[...] = xh
    c_ref[:, 0:H] = jnp.dot(xh, wd_ref[...], preferred_element_type=jnp.float32)
    c_ref[:, H:] = jnp.dot(xh, ws_ref[...], preferred_element_type=jnp.float32)


def _enc_nodes(x_feat, p, wd, ws):
    return pl.pallas_call(
        _enc_nodes_body,
        out_shape=[jax.ShapeDtypeStruct((N, H), jnp.float32),
                   jax.ShapeDtypeStruct((N, 2 * H), jnp.float32)],
    )(x_feat, p['W1'], p['b1'], p['W2'], p['b2'], p['g'], p['be'], wd, ws)


def _enc_edges_body(d_ref, w1_ref, b1_ref, w2_ref, b2_ref, g_ref, be_ref,
                    eh_ref):
    d = d_ref[...]
    rel = d[:, 0:3]
    relw = d[:, 3:6]
    dist = jnp.sqrt(jnp.sum(rel * rel, axis=-1, keepdims=True))
    distw = jnp.sqrt(jnp.sum(relw * relw, axis=-1, keepdims=True))
    w1 = w1_ref[...]
    acc = b1_ref[...] + jnp.zeros((d.shape[0], H), jnp.float32)
    for j, col in enumerate((d[:, 0:1], d[:, 1:2], d[:, 2:3], dist,
                             d[:, 3:4], d[:, 4:5], d[:, 5:6], distw,
                             d[:, 6:7])):
        acc = acc + col * w1[j:j + 1, :]
    h = jnp.maximum(acc, 0.0)
    eh_ref[...] = _ln(
        jnp.dot(h, w2_ref[...], preferred_element_type=jnp.float32)
        + b2_ref[...], g_ref[...], be_ref[...])


def _enc_edges(d128, p):
    ne = d128.shape[0]
    wfull = pl.BlockSpec((16, H), lambda i: (0, 0))
    wrow = pl.BlockSpec((1, H), lambda i: (0, 0))
    wsq = pl.BlockSpec((H, H), lambda i: (0, 0))
    return pl.pallas_call(
        _enc_edges_body,
        grid=(ne // BE,),
        in_specs=[pl.BlockSpec((BE, H), lambda i: (i, 0)),
                  wfull, wrow, wsq, wrow, wrow, wrow],
        out_specs=pl.BlockSpec((BE, H), lambda i: (i, 0)),
        out_shape=jax.ShapeDtypeStruct((ne, H), jnp.float32),
    )(d128, p['W1'], p['b1'], p['W2'], p['b2'], p['g'], p['be'])


def _edge_upd_body(eh_ref, pq_ref, we_ref, w2_ref, b1_ref, b2_ref,
                   g_ref, be_ref, msg_ref, eout_ref):
    eh = eh_ref[...]
    ec = jnp.dot(eh, we_ref[...], preferred_element_type=jnp.float32)
    b1 = b1_ref[...]
    g = g_ref[...]
    be = be_ref[...]
    w2 = w2_ref[...]
    b2 = b2_ref[...]
    hm = jnp.maximum(pq_ref[:, 0:H] + ec + b1, 0.0)
    msg_ref[...] = _ln(
        jnp.dot(hm, w2, preferred_element_type=jnp.float32) + b2, g, be)
    hn = jnp.maximum(pq_ref[:, H:] + ec + b1, 0.0)
    ne = _ln(jnp.dot(hn, w2, preferred_element_type=jnp.float32) + b2, g, be)
    eout_ref[...] = eh + ne


def _edge_update(eh, pq, we, w2, b1, b2, g, be):
    ne = eh.shape[0]
    blk = pl.BlockSpec((BE, H), lambda i: (i, 0))
    blk2 = pl.BlockSpec((BE, 2 * H), lambda i: (i, 0))
    wsq = pl.BlockSpec((H, H), lambda i: (0, 0))
    wrow = pl.BlockSpec((1, H), lambda i: (0, 0))
    f = jax.ShapeDtypeStruct((ne, H), jnp.float32)
    return pl.pallas_call(
        _edge_upd_body,
        grid=(ne // BE,),
        in_specs=[blk, blk2, wsq, wsq, wrow, wrow, wrow, wrow],
        out_specs=[blk, blk],
        out_shape=[f, f],
    )(eh, pq, we, w2, b1, b2, g, be)


def _node_upd_body(xh_ref, ag1_ref, ag2_ref, wa_ref, wx_ref, b1_ref, w2_ref,
                   b2_ref, g_ref, be_ref, wd_ref, ws_ref, xo_ref, c_ref):
    xh = xh_ref[...]
    ag = (ag1_ref[0] + ag1_ref[1]) + (ag2_ref[0] + ag2_ref[1])
    h = jnp.maximum(
        jnp.dot(ag, wa_ref[...], preferred_element_type=jnp.float32)
        + jnp.dot(xh, wx_ref[...], preferred_element_type=jnp.float32)
        + b1_ref[...], 0.0)
    nx = _ln(jnp.dot(h, w2_ref[...], preferred_element_type=jnp.float32)
             + b2_ref[...], g_ref[...], be_ref[...])
    xo = xh + nx
    xo_ref[...] = xo
    c_ref[:, 0:H] = jnp.dot(xo, wd_ref[...], preferred_element_type=jnp.float32)
    c_ref[:, H:] = jnp.dot(xo, ws_ref[...], preferred_element_type=jnp.float32)


def _node_update(xh, ag1, ag2, pn, wd, ws):
    return pl.pallas_call(
        _node_upd_body,
        out_shape=[jax.ShapeDtypeStruct((N, H), jnp.float32),
                   jax.ShapeDtypeStruct((N, 2 * H), jnp.float32)],
    )(xh, ag1, ag2, pn['W1'][:H], pn['W1'][H:], pn['b1'], pn['W2'], pn['b2'],
      pn['g'], pn['be'], wd, ws)


def _node_dec_body(xh_ref, ag1_ref, ag2_ref, wa_ref, wx_ref, b1_ref, w2_ref,
                   b2_ref, g_ref, be_ref, dw1_ref, db1_ref, dw2_ref, db2_ref,
                   y_ref):
    xh = xh_ref[...]
    ag = (ag1_ref[0] + ag1_ref[1]) + (ag2_ref[0] + ag2_ref[1])
    h = jnp.maximum(
        jnp.dot(ag, wa_ref[...], preferred_element_type=jnp.float32)
        + jnp.dot(xh, wx_ref[...], preferred_element_type=jnp.float32)
        + b1_ref[...], 0.0)
    nx = _ln(jnp.dot(h, w2_ref[...], preferred_element_type=jnp.float32)
             + b2_ref[...], g_ref[...], be_ref[...])
    xo = xh + nx
    dh = jnp.maximum(
        jnp.dot(xo, dw1_ref[...], preferred_element_type=jnp.float32)
        + db1_ref[...], 0.0)
    y_ref[...] = jnp.dot(dh, dw2_ref[...], preferred_element_type=jnp.float32) \
        + db2_ref[...]


def _node_dec(xh, ag1, ag2, pn, dec, dw2pad, db2pad):
    return pl.pallas_call(
        _node_dec_body,
        out_shape=jax.ShapeDtypeStruct((N, H), jnp.float32),
    )(xh, ag1, ag2, pn['W1'][:H], pn['W1'][H:], pn['b1'], pn['W2'], pn['b2'],
      pn['g'], pn['be'], dec['W1'], dec['b1'], dw2pad, db2pad)


def kernel(world_pos, mesh_pos, phi, swelling_phi, next_swelling_phi,
           rate_swelling_phi, node_type, mat_param, edge_index, params):
    src = edge_index[0]
    dst = edge_index[1]

    u = world_pos - mesh_pos
    mat = jnp.broadcast_to(mat_param[None, :], (N, 4))
    x_feat = jnp.concatenate(
        [u, phi, swelling_phi, next_swelling_phi, rate_swelling_phi,
         node_type, mat], axis=-1)
    x_feat = jnp.pad(x_feat, ((0, 0), (0, 12)))

    pe_n = params['node_enc']
    ne_p = {'W1': jnp.pad(pe_n['W1'], ((0, 12), (0, 0))),
            'b1': pe_n['b1'][None, :], 'W2': pe_n['W2'],
            'b2': pe_n['b2'][None, :], 'g': pe_n['g'][None, :],
            'be': pe_n['be'][None, :]}

    procs = params['procs']
    wd0 = procs[0]['edge']['W1'][:H]
    ws0 = procs[0]['edge']['W1'][H:2 * H]

    g128 = jnp.pad(jnp.concatenate([mesh_pos, world_pos, phi], axis=-1),
                   ((0, 0), (0, H - 7)))

    x_h, C = _enc_nodes(x_feat, ne_p, wd0, ws0)

    E2 = E // 2
    s1, s2 = src[:E2], src[E2:]
    d1, d2 = dst[:E2], dst[E2:]

    pe_e = params['edge_enc']
    ee_p = {'W1': jnp.pad(pe_e['W1'], ((0, 7), (0, 0))),
            'b1': pe_e['b1'][None, :], 'W2': pe_e['W2'],
            'b2': pe_e['b2'][None, :], 'g': pe_e['g'][None, :],
            'be': pe_e['be'][None, :]}

    dg1 = _sc_geom(g128, s1, d1)
    dg2 = _sc_geom(g128, s2, d2)
    eh1 = _enc_edges(dg1, ee_p)
    eh2 = _enc_edges(dg2, ee_p)

    for l in range(len(procs)):
        pe = procs[l]['edge']
        pn = procs[l]['node']
        we = pe['W1'][2 * H:]
        eb1 = pe['b1'][None, :]
        eb2 = pe['b2'][None, :]
        eg = pe['g'][None, :]
        ebe = pe['be'][None, :]
        pq1 = _sc_gather(C, s1, d1)
        pq2 = _sc_gather(C, s2, d2)
        msg1, eh1 = _edge_update(eh1, pq1, we, pe['W2'], eb1, eb2, eg, ebe)
        ag1 = _sc_scatter(msg1, d1)
        msg2, eh2 = _edge_update(eh2, pq2, we, pe['W2'], eb1, eb2, eg, ebe)
        ag2 = _sc_scatter(msg2, d2)
        pn_p = {'W1': pn['W1'], 'b1': pn['b1'][None, :], 'W2': pn['W2'],
                'b2': pn['b2'][None, :], 'g': pn['g'][None, :],
                'be': pn['be'][None, :]}
        if l + 1 < len(procs):
            wd = procs[l + 1]['edge']['W1'][:H]
            ws = procs[l + 1]['edge']['W1'][H:2 * H]
            x_h, C = _node_update(x_h, ag1, ag2, pn_p, wd, ws)
        else:
            dec = params['dec']
            dec_p = {'W1': dec['W1'], 'b1': dec['b1'][None, :]}
            dw2pad = jnp.pad(dec['W2'], ((0, 0), (0, H - 3)))
            db2pad = jnp.pad(dec['b2'], (0, H - 3))[None, :]
            y = _node_dec(x_h, ag1, ag2, pn_p, dec_p, dw2pad, db2pad)
    return y[:, :3]
